# baseline (speedup 1.0000x reference)
import functools

import jax
import jax.numpy as jnp
from jax import lax
from jax.experimental import pallas as pl
from jax.experimental.pallas import tpu as pltpu
from jax.experimental.pallas import tpu_sc as plsc

_NC = 2
_NS = 16
_NW = _NC * _NS
_CH = 4


@functools.lru_cache(maxsize=None)
def _make_gather(num_rows, dim, batch, seq):
    rows_per_w = batch // _NW
    n_chunks = rows_per_w // _CH
    mesh = plsc.VectorSubcoreMesh(core_axis_name="c", subcore_axis_name="s")

    @functools.partial(
        pl.kernel,
        mesh=mesh,
        out_type=jax.ShapeDtypeStruct((batch * seq, dim), jnp.float32),
        scratch_types=[
            pltpu.VMEM((rows_per_w * seq,), jnp.int32),
            pltpu.VMEM_SHARED((_NS * rows_per_w * seq,), jnp.int32),
            pltpu.SMEM((_CH * seq,), jnp.int32),
            pltpu.SMEM((_CH * seq,), jnp.int32),
            pltpu.VMEM((_CH * seq, dim), jnp.float32),
            pltpu.VMEM((_CH * seq, dim), jnp.float32),
            pltpu.SemaphoreType.DMA,
            pltpu.SemaphoreType.DMA,
            pltpu.SemaphoreType.DMA,
            pltpu.SemaphoreType.DMA,
        ],
    )
    def body(ids_hbm, table_hbm, out_hbm,
             ids_v, ids_sh, sm0, sm1, r0_v, r1_v, rsem0, rsem1, osem0, osem1):
        sm = (sm0, sm1)
        rbuf = (r0_v, r1_v)
        rsem = (rsem0, rsem1)
        osem = (osem0, osem1)
        sid = lax.axis_index("s")
        wid = sid * _NC + lax.axis_index("c")
        base = wid * rows_per_w
        cw = _CH * seq
        shbase = sid * (rows_per_w * seq)
        pltpu.sync_copy(ids_hbm.at[pl.ds(base * seq, rows_per_w * seq)], ids_v)
        pltpu.sync_copy(ids_v, ids_sh.at[pl.ds(shbase, rows_per_w * seq)])
        pltpu.sync_copy(ids_sh.at[pl.ds(shbase, cw)], sm0)
        pltpu.sync_copy(ids_sh.at[pl.ds(shbase + cw, cw)], sm1)

        def out_desc(c, b):
            return pltpu.make_async_copy(
                rbuf[b], out_hbm.at[pl.ds((base + c * _CH) * seq, cw)], osem[b])

        def row_drain_desc(c, b):
            return pltpu.make_async_copy(
                out_hbm.at[pl.ds((base + c * _CH) * seq, cw)], rbuf[b], rsem[b])

        @pl.loop(0, n_chunks, step=2)
        def grp(c0):
            for b in range(2):
                c = c0 + b

                @pl.when(c >= 2)
                def _():
                    out_desc(c - 2, b).wait()

                for i in range(cw):
                    tid = sm[b][i]
                    pltpu.make_async_copy(
                        table_hbm.at[tid], rbuf[b].at[i], rsem[b]
                    ).start()

                @pl.when(c + 2 < n_chunks)
                def _():
                    pltpu.sync_copy(
                        ids_sh.at[pl.ds(shbase + (c + 2) * cw, cw)], sm[b])

                row_drain_desc(c, b).wait()
                out_desc(c, b).start()

        out_desc(n_chunks - 2, 0).wait()
        out_desc(n_chunks - 1, 1).wait()

    return body


def kernel(token_ids, weight):
    batch, seq = token_ids.shape
    num_rows, dim = weight.shape
    ids = token_ids.astype(jnp.int32).reshape(-1)
    out = _make_gather(num_rows, dim, batch, seq)(ids, weight)
    return out.reshape(batch, seq, dim)


# speedup vs baseline: 1.1361x; 1.1361x over previous
import functools

import jax
import jax.numpy as jnp
from jax import lax
from jax.experimental import pallas as pl
from jax.experimental.pallas import tpu as pltpu
from jax.experimental.pallas import tpu_sc as plsc

_NC = 2
_NS = 16
_NW = _NC * _NS
_CH = 8


@functools.lru_cache(maxsize=None)
def _make_gather(num_rows, dim, batch, seq):
    rows_per_w = batch // _NW
    n_chunks = rows_per_w // _CH
    mesh = plsc.VectorSubcoreMesh(core_axis_name="c", subcore_axis_name="s")

    @functools.partial(
        pl.kernel,
        mesh=mesh,
        out_type=jax.ShapeDtypeStruct((batch, seq, dim), jnp.float32),
        scratch_types=[
            pltpu.VMEM((rows_per_w * seq,), jnp.int32),
            pltpu.VMEM_SHARED((_NS * rows_per_w * seq,), jnp.int32),
            pltpu.SMEM((_CH * seq,), jnp.int32),
            pltpu.SMEM((_CH * seq,), jnp.int32),
            pltpu.VMEM((_CH, seq, dim), jnp.float32),
            pltpu.VMEM((_CH, seq, dim), jnp.float32),
            pltpu.SemaphoreType.DMA,
            pltpu.SemaphoreType.DMA,
            pltpu.SemaphoreType.DMA,
            pltpu.SemaphoreType.DMA,
        ],
    )
    def body(ids_hbm, table_hbm, out_hbm,
             ids_v, ids_sh, sm0, sm1, r0_v, r1_v, rsem0, rsem1, osem0, osem1):
        sm = (sm0, sm1)
        rbuf = (r0_v, r1_v)
        rsem = (rsem0, rsem1)
        osem = (osem0, osem1)
        sid = lax.axis_index("s")
        wid = sid * _NC + lax.axis_index("c")
        base = wid * rows_per_w
        cw = _CH * seq
        shbase = sid * (rows_per_w * seq)
        pltpu.sync_copy(ids_hbm.at[pl.ds(base * seq, rows_per_w * seq)], ids_v)
        pltpu.sync_copy(ids_v, ids_sh.at[pl.ds(shbase, rows_per_w * seq)])
        pltpu.sync_copy(ids_sh.at[pl.ds(shbase, cw)], sm0)
        pltpu.sync_copy(ids_sh.at[pl.ds(shbase + cw, cw)], sm1)

        def out_desc(c, b):
            return pltpu.make_async_copy(
                rbuf[b], out_hbm.at[pl.ds(base + c * _CH, _CH)], osem[b])

        def row_drain_desc(c, b):
            return pltpu.make_async_copy(
                out_hbm.at[pl.ds(base + c * _CH, _CH)], rbuf[b], rsem[b])

        @pl.loop(0, n_chunks, step=2)
        def grp(c0):
            for b in range(2):
                c = c0 + b

                @pl.when(c >= 2)
                def _():
                    out_desc(c - 2, b).wait()

                for j in range(_CH):
                    for t in range(seq):
                        tid = sm[b][j * seq + t]
                        pltpu.make_async_copy(
                            table_hbm.at[tid], rbuf[b].at[j, t], rsem[b]
                        ).start()

                @pl.when(c + 2 < n_chunks)
                def _():
                    pltpu.sync_copy(
                        ids_sh.at[pl.ds(shbase + (c + 2) * cw, cw)], sm[b])

                row_drain_desc(c, b).wait()
                out_desc(c, b).start()

        out_desc(n_chunks - 2, 0).wait()
        out_desc(n_chunks - 1, 1).wait()

    return body


def kernel(token_ids, weight):
    batch, seq = token_ids.shape
    num_rows, dim = weight.shape
    ids = token_ids.astype(jnp.int32).reshape(-1)
    return _make_gather(num_rows, dim, batch, seq)(ids, weight)
